# Initial kernel scaffold; baseline (speedup 1.0000x reference)
import jax
import jax.numpy as jnp
from jax import lax
from jax.experimental import pallas as pl
from jax.experimental.pallas import tpu as pltpu
from jax.experimental.pallas import tpu_sc as plsc

N_ENVS = 4096
H = 64
W = 64
VIEW = 11
NC = 2
NS = 16
LANES = 16
NW = NC * NS
EPW = N_ENVS // NW
G = EPW // LANES
NROWS = 13
OUT_PER_ENV = 5 * VIEW * VIEW
PATCH = VIEW * VIEW


def _sc_body(grids_hbm, en_hbm, act_hbm, ax_hbm, ay_hbm, dx_hbm, dy_hbm,
             out_hbm, rows_v, out_v, en_v, act_v, ax_v, ay_v, dx_v, dy_v, sem):
    wid = lax.axis_index("s") * NC + lax.axis_index("c")
    base = pl.multiple_of(wid * EPW, EPW)

    pltpu.sync_copy(en_hbm.at[pl.ds(base, EPW)], en_v)
    pltpu.sync_copy(act_hbm.at[pl.ds(base, EPW)], act_v)
    pltpu.sync_copy(ax_hbm.at[pl.ds(base, EPW)], ax_v)
    pltpu.sync_copy(ay_hbm.at[pl.ds(base, EPW)], ay_v)
    pltpu.sync_copy(dx_hbm, dx_v)
    pltpu.sync_copy(dy_hbm, dy_v)

    lane = lax.iota(jnp.int32, LANES)
    outb0 = lane * OUT_PER_ENV

    @pl.loop(0, G)
    def _(g):
        off = g * LANES
        agx = ax_v[pl.ds(off, LANES)]
        agy = ay_v[pl.ds(off, LANES)]
        acts = act_v[pl.ds(off, LANES)]
        en = en_v[pl.ds(off, LANES)]

        ax = jnp.clip(agx, 1, W - 2)
        ay = jnp.clip(agy, 1, H - 2)
        dx = plsc.load_gather(dx_v, [acts])
        dy = plsc.load_gather(dy_v, [acts])
        nx = jnp.clip(ax + dx, 1, W - 2)
        ny = jnp.clip(ay + dy, 1, H - 2)

        rowb0 = (base + off + lane) * H
        copies = []
        for k in range(NROWS):
            ridx = rowb0 + jnp.clip(ay - 6 + k, 0, H - 1)
            copies.append(
                pltpu.async_copy(grids_hbm.at[ridx],
                                 rows_v.at[pl.ds(k * LANES, LANES)], sem))
        for cp in copies:
            cp.wait()

        tcf = plsc.load_gather(rows_v, [(ny - ay + 6) * LANES + lane, nx])
        blocked = (tcf * 4.0).astype(jnp.int32) == 1
        fx = jnp.where(blocked, ax, nx)
        fy = jnp.where(blocked, ay, ny)
        cur = plsc.load_gather(rows_v, [(fy - ay + 6) * LANES + lane, fx])
        lc = (cur * 4.0).astype(jnp.int32)
        food = lc == 2
        poison = lc == 3
        reward = jnp.where(food, 10.0, 0.0) - jnp.where(poison, 20.0, 0.0) - 0.1
        enc = (en + reward) / 100.0
        centerval = jnp.where(food | poison, 0.0, cur)

        rowbase0 = (fy - ay + 1) * LANES + lane
        col0 = fx - 5

        @pl.loop(0, VIEW)
        def _(k):
            rowb = rowbase0 + k * LANES
            row = fy + (k - 5)
            rin = (row >= 0) & (row <= H - 1)
            ob_k = outb0 + k * VIEW
            for j in range(VIEW):
                col = col0 + j
                inb = rin & (col >= 0) & (col <= W - 1)
                colc = jnp.clip(col, 0, W - 1)
                v = plsc.load_gather(rows_v, [rowb, colc])
                patch = jnp.where(inb, v, 0.3)
                cell = (patch * 4.0).astype(jnp.int32)
                o0 = ob_k + j
                plsc.store_scatter(out_v, [o0], patch)
                plsc.store_scatter(out_v, [o0 + PATCH],
                                   jnp.where(cell == 1, 1.0, 0.0))
                plsc.store_scatter(out_v, [o0 + 2 * PATCH],
                                   jnp.where(cell == 2, 1.0, 0.0))
                plsc.store_scatter(out_v, [o0 + 3 * PATCH],
                                   jnp.where(cell == 3, 1.0, 0.0))
                plsc.store_scatter(out_v, [o0 + 4 * PATCH], enc)

        ccell = (centerval * 4.0).astype(jnp.int32)
        oc = outb0 + 5 * VIEW + 5
        plsc.store_scatter(out_v, [oc], centerval)
        plsc.store_scatter(out_v, [oc + PATCH],
                           jnp.where(ccell == 1, 1.0, 0.0))
        plsc.store_scatter(out_v, [oc + 2 * PATCH],
                           jnp.where(ccell == 2, 1.0, 0.0))
        plsc.store_scatter(out_v, [oc + 3 * PATCH],
                           jnp.where(ccell == 3, 1.0, 0.0))

        ooff = pl.multiple_of((base + off) * OUT_PER_ENV, 8)
        pltpu.sync_copy(out_v, out_hbm.at[pl.ds(ooff, LANES * OUT_PER_ENV)])


def kernel(grids, agent_energy, actions, agent_x, agent_y):
    grids2d = grids.reshape(N_ENVS * H, W)
    dx16 = jnp.array([0, 0, 0, -1, 1, -1, -1, 1, 1, 0, 0, 0, 0, 0, 0, 0],
                     jnp.int32)
    dy16 = jnp.array([0, -1, 1, 0, 0, -1, 1, -1, 1, 0, 0, 0, 0, 0, 0, 0],
                     jnp.int32)

    sc_fn = pl.kernel(
        _sc_body,
        out_type=jax.ShapeDtypeStruct((N_ENVS * OUT_PER_ENV,), jnp.float32),
        mesh=plsc.VectorSubcoreMesh(core_axis_name="c", subcore_axis_name="s"),
        scratch_types=[
            pltpu.VMEM((NROWS * LANES, W), jnp.float32),
            pltpu.VMEM((LANES * OUT_PER_ENV,), jnp.float32),
            pltpu.VMEM((EPW,), jnp.float32),
            pltpu.VMEM((EPW,), jnp.int32),
            pltpu.VMEM((EPW,), jnp.int32),
            pltpu.VMEM((EPW,), jnp.int32),
            pltpu.VMEM((LANES,), jnp.int32),
            pltpu.VMEM((LANES,), jnp.int32),
            pltpu.SemaphoreType.DMA,
        ],
    )
    flat = sc_fn(grids2d, agent_energy,
                 actions.astype(jnp.int32),
                 agent_x.astype(jnp.int32),
                 agent_y.astype(jnp.int32),
                 dx16, dy16)
    return flat.reshape(N_ENVS, 5, VIEW, VIEW)


# speedup vs baseline: 2.1427x; 2.1427x over previous
import jax
import jax.numpy as jnp
from jax import lax
from jax.experimental import pallas as pl
from jax.experimental.pallas import tpu as pltpu
from jax.experimental.pallas import tpu_sc as plsc

N_ENVS = 4096
H = 64
W = 64
VIEW = 11
NC = 2
NS = 16
LANES = 16
NW = NC * NS
EPW = N_ENVS // NW
G = EPW // LANES
NROWS = 13
OUT_PER_ENV = 5 * VIEW * VIEW
PATCH = VIEW * VIEW


def _sc_body(grids_hbm, en_hbm, act_hbm, ax_hbm, ay_hbm, dx_hbm, dy_hbm,
             out_hbm, rows_v, out_v, idx_v, en_v, act_v, ax_v, ay_v, dx_v,
             dy_v, sem):
    wid = lax.axis_index("s") * NC + lax.axis_index("c")
    base = pl.multiple_of(wid * EPW, EPW)

    pltpu.sync_copy(en_hbm.at[pl.ds(base, EPW)], en_v)
    pltpu.sync_copy(act_hbm.at[pl.ds(base, EPW)], act_v)
    pltpu.sync_copy(ax_hbm.at[pl.ds(base, EPW)], ax_v)
    pltpu.sync_copy(ay_hbm.at[pl.ds(base, EPW)], ay_v)
    pltpu.sync_copy(dx_hbm, dx_v)
    pltpu.sync_copy(dy_hbm, dy_v)

    lane = lax.iota(jnp.int32, LANES)
    outb0 = lane * OUT_PER_ENV

    @pl.loop(0, G)
    def _(g):
        off = g * LANES
        agx = ax_v[pl.ds(off, LANES)]
        agy = ay_v[pl.ds(off, LANES)]
        acts = act_v[pl.ds(off, LANES)]
        en = en_v[pl.ds(off, LANES)]

        ax = jnp.clip(agx, 1, W - 2)
        ay = jnp.clip(agy, 1, H - 2)
        dx = plsc.load_gather(dx_v, [acts])
        dy = plsc.load_gather(dy_v, [acts])
        nx = jnp.clip(ax + dx, 1, W - 2)
        ny = jnp.clip(ay + dy, 1, H - 2)

        rowb0 = (base + off + lane) * H
        for k in range(NROWS):
            idx_v[pl.ds(k * LANES, LANES)] = rowb0 + jnp.clip(ay - 6 + k, 0,
                                                              H - 1)
        half = (NROWS * LANES) // 2
        cp0 = pltpu.async_copy(grids_hbm.at[idx_v.at[pl.ds(0, half)]],
                               rows_v.at[pl.ds(0, half)], sem)
        cp1 = pltpu.async_copy(grids_hbm.at[idx_v.at[pl.ds(half, half)]],
                               rows_v.at[pl.ds(half, half)], sem)
        cp0.wait()
        cp1.wait()

        tcf = plsc.load_gather(rows_v, [(ny - ay + 6) * LANES + lane, nx])
        blocked = (tcf * 4.0).astype(jnp.int32) == 1
        fx = jnp.where(blocked, ax, nx)
        fy = jnp.where(blocked, ay, ny)
        cur = plsc.load_gather(rows_v, [(fy - ay + 6) * LANES + lane, fx])
        lc = (cur * 4.0).astype(jnp.int32)
        food = lc == 2
        poison = lc == 3
        reward = jnp.where(food, 10.0, 0.0) - jnp.where(poison, 20.0, 0.0) - 0.1
        enc = (en + reward) / 100.0
        centerval = jnp.where(food | poison, 0.0, cur)

        rowbase0 = (fy - ay + 1) * LANES + lane
        col0 = fx - 5

        @pl.loop(0, VIEW)
        def _(k):
            rowb = rowbase0 + k * LANES
            row = fy + (k - 5)
            rin = (row >= 0) & (row <= H - 1)
            ob_k = outb0 + k * VIEW
            for j in range(VIEW):
                col = col0 + j
                inb = rin & (col >= 0) & (col <= W - 1)
                colc = jnp.clip(col, 0, W - 1)
                v = plsc.load_gather(rows_v, [rowb, colc])
                patch = jnp.where(inb, v, 0.3)
                cell = (patch * 4.0).astype(jnp.int32)
                o0 = ob_k + j
                plsc.store_scatter(out_v, [o0], patch)
                plsc.store_scatter(out_v, [o0 + PATCH],
                                   jnp.where(cell == 1, 1.0, 0.0))
                plsc.store_scatter(out_v, [o0 + 2 * PATCH],
                                   jnp.where(cell == 2, 1.0, 0.0))
                plsc.store_scatter(out_v, [o0 + 3 * PATCH],
                                   jnp.where(cell == 3, 1.0, 0.0))
                plsc.store_scatter(out_v, [o0 + 4 * PATCH], enc)

        ccell = (centerval * 4.0).astype(jnp.int32)
        oc = outb0 + 5 * VIEW + 5
        plsc.store_scatter(out_v, [oc], centerval)
        plsc.store_scatter(out_v, [oc + PATCH],
                           jnp.where(ccell == 1, 1.0, 0.0))
        plsc.store_scatter(out_v, [oc + 2 * PATCH],
                           jnp.where(ccell == 2, 1.0, 0.0))
        plsc.store_scatter(out_v, [oc + 3 * PATCH],
                           jnp.where(ccell == 3, 1.0, 0.0))

        ooff = pl.multiple_of((base + off) * OUT_PER_ENV, 8)
        pltpu.sync_copy(out_v, out_hbm.at[pl.ds(ooff, LANES * OUT_PER_ENV)])


def kernel(grids, agent_energy, actions, agent_x, agent_y):
    grids2d = grids.reshape(N_ENVS * H, W)
    dx16 = jnp.array([0, 0, 0, -1, 1, -1, -1, 1, 1, 0, 0, 0, 0, 0, 0, 0],
                     jnp.int32)
    dy16 = jnp.array([0, -1, 1, 0, 0, -1, 1, -1, 1, 0, 0, 0, 0, 0, 0, 0],
                     jnp.int32)

    sc_fn = pl.kernel(
        _sc_body,
        out_type=jax.ShapeDtypeStruct((N_ENVS * OUT_PER_ENV,), jnp.float32),
        mesh=plsc.VectorSubcoreMesh(core_axis_name="c", subcore_axis_name="s"),
        compiler_params=pltpu.CompilerParams(needs_layout_passes=False,
                                             use_tc_tiling_on_sc=False),
        scratch_types=[
            pltpu.VMEM((NROWS * LANES, W), jnp.float32),
            pltpu.VMEM((LANES * OUT_PER_ENV,), jnp.float32),
            pltpu.VMEM((NROWS * LANES,), jnp.int32),
            pltpu.VMEM((EPW,), jnp.float32),
            pltpu.VMEM((EPW,), jnp.int32),
            pltpu.VMEM((EPW,), jnp.int32),
            pltpu.VMEM((EPW,), jnp.int32),
            pltpu.VMEM((LANES,), jnp.int32),
            pltpu.VMEM((LANES,), jnp.int32),
            pltpu.SemaphoreType.DMA,
        ],
    )
    flat = sc_fn(grids2d, agent_energy,
                 actions.astype(jnp.int32),
                 agent_x.astype(jnp.int32),
                 agent_y.astype(jnp.int32),
                 dx16, dy16)
    return flat.reshape(N_ENVS, 5, VIEW, VIEW)


# speedup vs baseline: 4.0258x; 1.8788x over previous
import jax
import jax.numpy as jnp
from jax import lax
from jax.experimental import pallas as pl
from jax.experimental.pallas import tpu as pltpu
from jax.experimental.pallas import tpu_sc as plsc

N_ENVS = 4096
H = 64
W = 64
VIEW = 11
NC = 2
NS = 16
LANES = 16
NW = NC * NS
EPW = N_ENVS // NW
G = EPW // LANES
NROWS = 13
OUT_PER_ENV = 5 * VIEW * VIEW
PATCH = VIEW * VIEW


def _sc_body(grids_hbm, en_hbm, act_hbm, ax_hbm, ay_hbm, dx_hbm, dy_hbm,
             out_hbm, rows_v, out_v, idx_v, en_v, act_v, ax_v, ay_v, dx_v,
             dy_v, sem):
    wid = lax.axis_index("s") * NC + lax.axis_index("c")
    base = pl.multiple_of(wid * EPW, EPW)

    pltpu.sync_copy(en_hbm.at[pl.ds(base, EPW)], en_v)
    pltpu.sync_copy(act_hbm.at[pl.ds(base, EPW)], act_v)
    pltpu.sync_copy(ax_hbm.at[pl.ds(base, EPW)], ax_v)
    pltpu.sync_copy(ay_hbm.at[pl.ds(base, EPW)], ay_v)
    pltpu.sync_copy(dx_hbm, dx_v)
    pltpu.sync_copy(dy_hbm, dy_v)

    lane = lax.iota(jnp.int32, LANES)

    @pl.loop(0, G)
    def _(g):
        off = g * LANES
        agx = ax_v[pl.ds(off, LANES)]
        agy = ay_v[pl.ds(off, LANES)]
        acts = act_v[pl.ds(off, LANES)]
        en = en_v[pl.ds(off, LANES)]

        ax = jnp.clip(agx, 1, W - 2)
        ay = jnp.clip(agy, 1, H - 2)
        dx = plsc.load_gather(dx_v, [acts])
        dy = plsc.load_gather(dy_v, [acts])
        nx = jnp.clip(ax + dx, 1, W - 2)
        ny = jnp.clip(ay + dy, 1, H - 2)

        rowb0 = (base + off + lane) * H
        for k in range(NROWS):
            idx_v[pl.ds(k * LANES, LANES)] = rowb0 + jnp.clip(ay - 6 + k, 0,
                                                              H - 1)
        half = (NROWS * LANES) // 2
        cp0 = pltpu.async_copy(grids_hbm.at[idx_v.at[pl.ds(0, half)]],
                               rows_v.at[pl.ds(0, half)], sem)
        cp1 = pltpu.async_copy(grids_hbm.at[idx_v.at[pl.ds(half, half)]],
                               rows_v.at[pl.ds(half, half)], sem)
        cp0.wait()
        cp1.wait()

        tcf = plsc.load_gather(rows_v, [(ny - ay + 6) * LANES + lane, nx])
        blocked = (tcf * 4.0).astype(jnp.int32) == 1
        fx = jnp.where(blocked, ax, nx)
        fy = jnp.where(blocked, ay, ny)
        cur = plsc.load_gather(rows_v, [(fy - ay + 6) * LANES + lane, fx])
        lc = (cur * 4.0).astype(jnp.int32)
        food = lc == 2
        poison = lc == 3
        reward = jnp.where(food, 10.0, 0.0) - jnp.where(poison, 20.0, 0.0) - 0.1
        enc = (en + reward) / 100.0
        centerval = jnp.where(food | poison, 0.0, cur)

        rowbase0 = (fy - ay + 1) * LANES + lane
        col0 = fx - 5

        @pl.loop(0, VIEW)
        def _(k):
            rowb = rowbase0 + k * LANES
            row = fy + (k - 5)
            rin = (row >= 0) & (row <= H - 1)
            for j in range(VIEW):
                p = k * VIEW + j
                col = col0 + j
                inb = rin & (col >= 0) & (col <= W - 1)
                colc = jnp.clip(col, 0, W - 1)
                v = plsc.load_gather(rows_v, [rowb, colc])
                patch = jnp.where(inb, v, 0.3)
                cell = (patch * 4.0).astype(jnp.int32)
                out_v[p, :] = patch
                out_v[PATCH + p, :] = jnp.where(cell == 1, 1.0, 0.0)
                out_v[2 * PATCH + p, :] = jnp.where(cell == 2, 1.0, 0.0)
                out_v[3 * PATCH + p, :] = jnp.where(cell == 3, 1.0, 0.0)
                out_v[4 * PATCH + p, :] = enc

        ccell = (centerval * 4.0).astype(jnp.int32)
        oc = 5 * VIEW + 5
        out_v[oc, :] = centerval
        out_v[PATCH + oc, :] = jnp.where(ccell == 1, 1.0, 0.0)
        out_v[2 * PATCH + oc, :] = jnp.where(ccell == 2, 1.0, 0.0)
        out_v[3 * PATCH + oc, :] = jnp.where(ccell == 3, 1.0, 0.0)

        e0 = pl.multiple_of(base + off, LANES)
        pltpu.sync_copy(out_v, out_hbm.at[:, pl.ds(e0, LANES)])


def kernel(grids, agent_energy, actions, agent_x, agent_y):
    grids2d = grids.reshape(N_ENVS * H, W)
    dx16 = jnp.array([0, 0, 0, -1, 1, -1, -1, 1, 1, 0, 0, 0, 0, 0, 0, 0],
                     jnp.int32)
    dy16 = jnp.array([0, -1, 1, 0, 0, -1, 1, -1, 1, 0, 0, 0, 0, 0, 0, 0],
                     jnp.int32)

    sc_fn = pl.kernel(
        _sc_body,
        out_type=jax.ShapeDtypeStruct((OUT_PER_ENV, N_ENVS), jnp.float32),
        mesh=plsc.VectorSubcoreMesh(core_axis_name="c", subcore_axis_name="s"),
        compiler_params=pltpu.CompilerParams(needs_layout_passes=False,
                                             use_tc_tiling_on_sc=False),
        scratch_types=[
            pltpu.VMEM((NROWS * LANES, W), jnp.float32),
            pltpu.VMEM((OUT_PER_ENV, LANES), jnp.float32),
            pltpu.VMEM((NROWS * LANES,), jnp.int32),
            pltpu.VMEM((EPW,), jnp.float32),
            pltpu.VMEM((EPW,), jnp.int32),
            pltpu.VMEM((EPW,), jnp.int32),
            pltpu.VMEM((EPW,), jnp.int32),
            pltpu.VMEM((LANES,), jnp.int32),
            pltpu.VMEM((LANES,), jnp.int32),
            pltpu.SemaphoreType.DMA,
        ],
    )
    flat = sc_fn(grids2d, agent_energy,
                 actions.astype(jnp.int32),
                 agent_x.astype(jnp.int32),
                 agent_y.astype(jnp.int32),
                 dx16, dy16)
    return flat.reshape(5, VIEW, VIEW, N_ENVS).transpose(3, 0, 1, 2)
